# initial kernel scaffold (unmeasured)
import jax
import jax.numpy as jnp
from jax import lax
from jax.experimental import pallas as pl
from jax.experimental.pallas import tpu as pltpu


def kernel(
    x,
):
    def body(*refs):
        pass

    out_shape = jax.ShapeDtypeStruct(..., jnp.float32)
    return pl.pallas_call(body, out_shape=out_shape)(...)


# baseline (device time: 81847 ns/iter reference)
import jax
import jax.numpy as jnp
from jax import lax
from jax.experimental import pallas as pl
from jax.experimental.pallas import tpu as pltpu

Y = 4
M = 1024
N_TOT = 2048
N_OUT = N_TOT // Y


def kernel(x):

    def body(x_ref, out_ref, send_buf, recv_buf, send_sems, recv_sems):
        my_x = lax.axis_index("x")
        my_y = lax.axis_index("y")
        my_z = lax.axis_index("z")
        left = (my_y - 1) % Y
        right = (my_y + 1) % Y

        barrier_sem = pltpu.get_barrier_semaphore()
        for nbr in (left, right):
            pl.semaphore_signal(
                barrier_sem, inc=1,
                device_id=(my_x, nbr, my_z),
                device_id_type=pl.DeviceIdType.MESH,
            )
        pl.semaphore_wait(barrier_sem, 2)

        for s in range(Y - 1):
            c = (my_y - 1 - s) % Y
            col = pl.ds(c * N_OUT, N_OUT)
            if s == 0:
                send_buf[s] = x_ref[0, :, col]
            else:
                send_buf[s] = recv_buf[s - 1] + x_ref[0, :, col]
            rdma = pltpu.make_async_remote_copy(
                src_ref=send_buf.at[s],
                dst_ref=recv_buf.at[s],
                send_sem=send_sems.at[s],
                recv_sem=recv_sems.at[s],
                device_id=(my_x, right, my_z),
                device_id_type=pl.DeviceIdType.MESH,
            )
            rdma.start()
            rdma.wait()

        out_ref[...] = recv_buf[Y - 2] + x_ref[0, :, pl.ds(my_y * N_OUT, N_OUT)]

    return pl.pallas_call(
        body,
        out_shape=jax.ShapeDtypeStruct((M, N_OUT), jnp.float32),
        in_specs=[pl.BlockSpec(memory_space=pltpu.VMEM)],
        out_specs=pl.BlockSpec(memory_space=pltpu.VMEM),
        scratch_shapes=[
            pltpu.VMEM((Y - 1, M, N_OUT), jnp.float32),
            pltpu.VMEM((Y - 1, M, N_OUT), jnp.float32),
            pltpu.SemaphoreType.DMA((Y - 1,)),
            pltpu.SemaphoreType.DMA((Y - 1,)),
        ],
        compiler_params=pltpu.CompilerParams(collective_id=0),
    )(x)


# device time: 53117 ns/iter; 1.5409x vs baseline; 1.5409x over previous
import jax
import jax.numpy as jnp
from jax import lax
from jax.experimental import pallas as pl
from jax.experimental.pallas import tpu as pltpu

Y = 4
Z = 4
M = 1024
N_TOT = 2048
N_OUT = N_TOT // Y
NQ = N_OUT // Z


def kernel(x):

    def body(x_ref, out_ref, rs_send, rs_recv, ag_buf,
             rs_send_sems, rs_recv_sems, ag_send_sems, ag_recv_sems):
        my_x = lax.axis_index("x")
        my_y = lax.axis_index("y")
        my_z = lax.axis_index("z")
        left_y = (my_y - 1) % Y
        right_y = (my_y + 1) % Y
        left_z = (my_z - 1) % Z
        right_z = (my_z + 1) % Z

        barrier_sem = pltpu.get_barrier_semaphore()
        for nbr in ((my_x, left_y, my_z), (my_x, right_y, my_z),
                    (my_x, my_y, left_z), (my_x, my_y, right_z)):
            pl.semaphore_signal(
                barrier_sem, inc=1,
                device_id=nbr, device_id_type=pl.DeviceIdType.MESH,
            )
        pl.semaphore_wait(barrier_sem, 4)

        for s in range(Y - 1):
            c = (my_y - 1 - s) % Y
            col = pl.ds(c * N_OUT + my_z * NQ, NQ)
            if s == 0:
                rs_send[s] = x_ref[0, :, col]
            else:
                rs_send[s] = rs_recv[s - 1] + x_ref[0, :, col]
            rdma = pltpu.make_async_remote_copy(
                src_ref=rs_send.at[s],
                dst_ref=rs_recv.at[s],
                send_sem=rs_send_sems.at[s],
                recv_sem=rs_recv_sems.at[s],
                device_id=(my_x, right_y, my_z),
                device_id_type=pl.DeviceIdType.MESH,
            )
            rdma.start()
            rdma.wait()

        my_quarter = rs_recv[Y - 2] + x_ref[
            0, :, pl.ds(my_y * N_OUT + my_z * NQ, NQ)]
        ag_buf[0] = my_quarter
        out_ref[:, pl.ds(my_z * NQ, NQ)] = my_quarter

        for h in range(Z - 1):
            rdma = pltpu.make_async_remote_copy(
                src_ref=ag_buf.at[h],
                dst_ref=ag_buf.at[h + 1],
                send_sem=ag_send_sems.at[h],
                recv_sem=ag_recv_sems.at[h],
                device_id=(my_x, my_y, right_z),
                device_id_type=pl.DeviceIdType.MESH,
            )
            rdma.start()
            rdma.wait()
            origin = (my_z - h - 1) % Z
            out_ref[:, pl.ds(origin * NQ, NQ)] = ag_buf[h + 1]

    return pl.pallas_call(
        body,
        out_shape=jax.ShapeDtypeStruct((M, N_OUT), jnp.float32),
        in_specs=[pl.BlockSpec(memory_space=pltpu.VMEM)],
        out_specs=pl.BlockSpec(memory_space=pltpu.VMEM),
        scratch_shapes=[
            pltpu.VMEM((Y - 1, M, NQ), jnp.float32),
            pltpu.VMEM((Y - 1, M, NQ), jnp.float32),
            pltpu.VMEM((Z, M, NQ), jnp.float32),
            pltpu.SemaphoreType.DMA((Y - 1,)),
            pltpu.SemaphoreType.DMA((Y - 1,)),
            pltpu.SemaphoreType.DMA((Z - 1,)),
            pltpu.SemaphoreType.DMA((Z - 1,)),
        ],
        compiler_params=pltpu.CompilerParams(collective_id=0),
    )(x)


# device time: 45689 ns/iter; 1.7914x vs baseline; 1.1626x over previous
import jax
import jax.numpy as jnp
from jax import lax
from jax.experimental import pallas as pl
from jax.experimental.pallas import tpu as pltpu

Y = 4
Z = 4
M = 1024
N_TOT = 2048
N_OUT = N_TOT // Y
NQ = N_OUT // Z
P = 2
RP = M // P


def kernel(x):

    def body(x_ref, out_ref, rs_send, rs_recv, ag_buf,
             rs_send_sems, rs_recv_sems, ag_send_sems, ag_recv_sems):
        my_x = lax.axis_index("x")
        my_y = lax.axis_index("y")
        my_z = lax.axis_index("z")
        left_y = (my_y - 1) % Y
        right_y = (my_y + 1) % Y
        left_z = (my_z - 1) % Z
        right_z = (my_z + 1) % Z

        barrier_sem = pltpu.get_barrier_semaphore()
        for nbr in ((my_x, left_y, my_z), (my_x, right_y, my_z),
                    (my_x, my_y, left_z), (my_x, my_y, right_z)):
            pl.semaphore_signal(
                barrier_sem, inc=1,
                device_id=nbr, device_id_type=pl.DeviceIdType.MESH,
            )
        pl.semaphore_wait(barrier_sem, 4)

        def rs_rdma(p, s):
            return pltpu.make_async_remote_copy(
                src_ref=rs_send.at[p, s],
                dst_ref=rs_recv.at[p, s],
                send_sem=rs_send_sems.at[p, s],
                recv_sem=rs_recv_sems.at[p, s],
                device_id=(my_x, right_y, my_z),
                device_id_type=pl.DeviceIdType.MESH,
            )

        def ag_rdma(p, h):
            return pltpu.make_async_remote_copy(
                src_ref=ag_buf.at[p, h],
                dst_ref=ag_buf.at[p, h + 1],
                send_sem=ag_send_sems.at[p, h],
                recv_sem=ag_recv_sems.at[p, h],
                device_id=(my_x, my_y, right_z),
                device_id_type=pl.DeviceIdType.MESH,
            )

        rs_h = [[None] * (Y - 1) for _ in range(P)]
        for s in range(Y - 1):
            c = (my_y - 1 - s) % Y
            for p in range(P):
                rows = pl.ds(p * RP, RP)
                col = pl.ds(c * N_OUT + my_z * NQ, NQ)
                if s == 0:
                    rs_send[p, s] = x_ref[0, rows, col]
                else:
                    rs_h[p][s - 1].wait()
                    rs_send[p, s] = rs_recv[p, s - 1] + x_ref[0, rows, col]
                rs_h[p][s] = rs_rdma(p, s)
                rs_h[p][s].start()

        ag_h = [[None] * (Z - 1) for _ in range(P)]
        for p in range(P):
            rs_h[p][Y - 2].wait()
            piece = rs_recv[p, Y - 2] + x_ref[
                0, pl.ds(p * RP, RP), pl.ds(my_y * N_OUT + my_z * NQ, NQ)]
            ag_buf[p, 0] = piece
            out_ref[pl.ds(p * RP, RP), pl.ds(my_z * NQ, NQ)] = piece
            ag_h[p][0] = ag_rdma(p, 0)
            ag_h[p][0].start()

        for h in range(1, Z - 1):
            origin = (my_z - h) % Z
            for p in range(P):
                ag_h[p][h - 1].wait()
                out_ref[pl.ds(p * RP, RP), pl.ds(origin * NQ, NQ)] = ag_buf[p, h]
                ag_h[p][h] = ag_rdma(p, h)
                ag_h[p][h].start()

        origin = (my_z - (Z - 1)) % Z
        for p in range(P):
            ag_h[p][Z - 2].wait()
            out_ref[pl.ds(p * RP, RP), pl.ds(origin * NQ, NQ)] = ag_buf[p, Z - 1]

    return pl.pallas_call(
        body,
        out_shape=jax.ShapeDtypeStruct((M, N_OUT), jnp.float32),
        in_specs=[pl.BlockSpec(memory_space=pltpu.VMEM)],
        out_specs=pl.BlockSpec(memory_space=pltpu.VMEM),
        scratch_shapes=[
            pltpu.VMEM((P, Y - 1, RP, NQ), jnp.float32),
            pltpu.VMEM((P, Y - 1, RP, NQ), jnp.float32),
            pltpu.VMEM((P, Z, RP, NQ), jnp.float32),
            pltpu.SemaphoreType.DMA((P, Y - 1)),
            pltpu.SemaphoreType.DMA((P, Y - 1)),
            pltpu.SemaphoreType.DMA((P, Z - 1)),
            pltpu.SemaphoreType.DMA((P, Z - 1)),
        ],
        compiler_params=pltpu.CompilerParams(collective_id=0),
    )(x)


# device time: 34022 ns/iter; 2.4057x vs baseline; 1.3429x over previous
import jax
import jax.numpy as jnp
from jax import lax
from jax.experimental import pallas as pl
from jax.experimental.pallas import tpu as pltpu

Y = 4
Z = 4
M = 1024
N_TOT = 2048
N_OUT = N_TOT // Y
NQ = N_OUT // Z
P = 4
RH = M // 2
RP = RH // P


def kernel(x):

    def body(x_ref, out_ref, rs_send, rs_recv,
             rs_send_sems, rs_recv_sems, ag_send_sems, ag_recv_sems,
             xs_send_sems, xs_recv_sems):
        my_x = lax.axis_index("x")
        my_y = lax.axis_index("y")
        my_z = lax.axis_index("z")
        left_y = (my_y - 1) % Y
        right_y = (my_y + 1) % Y
        right_z = (my_z + 1) % Z
        left_z = (my_z - 1) % Z
        other_x = 1 - my_x

        def my_rows(p):
            return pl.ds(my_x * RH + p * RP, RP)

        def partner_rows(p):
            return pl.ds(other_x * RH + p * RP, RP)

        barrier_sem = pltpu.get_barrier_semaphore()
        for nbr in ((my_x, left_y, my_z), (my_x, right_y, my_z),
                    (my_x, my_y, left_z), (my_x, my_y, right_z),
                    (other_x, my_y, my_z)):
            pl.semaphore_signal(
                barrier_sem, inc=1,
                device_id=nbr, device_id_type=pl.DeviceIdType.MESH,
            )
        pl.semaphore_wait(barrier_sem, 5)

        def rs_rdma(p, s):
            return pltpu.make_async_remote_copy(
                src_ref=rs_send.at[p, s],
                dst_ref=rs_recv.at[p, s],
                send_sem=rs_send_sems.at[p, s],
                recv_sem=rs_recv_sems.at[p, s],
                device_id=(my_x, right_y, my_z),
                device_id_type=pl.DeviceIdType.MESH,
            )

        def out_slice(p, origin):
            return out_ref.at[my_rows(p), pl.ds((origin % Z) * NQ, NQ)]

        def ag_send(p, h):
            sl = out_slice(p, my_z - h)
            return pltpu.make_async_remote_copy(
                src_ref=sl, dst_ref=sl,
                send_sem=ag_send_sems.at[p, h],
                recv_sem=ag_recv_sems.at[p, h],
                device_id=(my_x, my_y, right_z),
                device_id_type=pl.DeviceIdType.MESH,
            )

        def ag_recv(p, h):
            sl = out_slice(p, my_z - h - 1)
            return pltpu.make_async_remote_copy(
                src_ref=sl, dst_ref=sl,
                send_sem=ag_send_sems.at[p, h],
                recv_sem=ag_recv_sems.at[p, h],
                device_id=(my_x, my_y, right_z),
                device_id_type=pl.DeviceIdType.MESH,
            )

        def xs_ship(p, k, origin):
            sl = out_slice(p, origin)
            return pltpu.make_async_remote_copy(
                src_ref=sl, dst_ref=sl,
                send_sem=xs_send_sems.at[p, k],
                recv_sem=xs_recv_sems.at[p, k],
                device_id=(other_x, my_y, my_z),
                device_id_type=pl.DeviceIdType.MESH,
            )

        def xs_recv(p, k, origin):
            sl = out_ref.at[partner_rows(p), pl.ds((origin % Z) * NQ, NQ)]
            return pltpu.make_async_remote_copy(
                src_ref=sl, dst_ref=sl,
                send_sem=xs_send_sems.at[p, k],
                recv_sem=xs_recv_sems.at[p, k],
                device_id=(other_x, my_y, my_z),
                device_id_type=pl.DeviceIdType.MESH,
            )

        rs_h = [[None] * (Y - 1) for _ in range(P)]
        for s in range(Y - 1):
            c = (my_y - 1 - s) % Y
            for p in range(P):
                col = pl.ds(c * N_OUT + my_z * NQ, NQ)
                if s == 0:
                    rs_send[p, s] = x_ref[0, my_rows(p), col]
                else:
                    rs_h[p][s - 1].wait()
                    rs_send[p, s] = rs_recv[p, s - 1] + x_ref[0, my_rows(p), col]
                rs_h[p][s] = rs_rdma(p, s)
                rs_h[p][s].start()

        started = []
        ag_send_h = [[None] * (Z - 1) for _ in range(P)]
        for p in range(P):
            rs_h[p][Y - 2].wait()
            out_ref[my_rows(p), pl.ds(my_z * NQ, NQ)] = rs_recv[p, Y - 2] + x_ref[
                0, my_rows(p), pl.ds(my_y * N_OUT + my_z * NQ, NQ)]
            ag_send_h[p][0] = ag_send(p, 0)
            ag_send_h[p][0].start()
            ship = xs_ship(p, 0, my_z)
            ship.start()
            started.append(ship)

        for h in range(Z - 1):
            for p in range(P):
                ag_recv(p, h).wait_recv()
                if h + 1 < Z - 1:
                    ag_send_h[p][h + 1] = ag_send(p, h + 1)
                    ag_send_h[p][h + 1].start()
                ship = xs_ship(p, h + 1, my_z - h - 1)
                ship.start()
                started.append(ship)

        for p in range(P):
            for h in range(Z - 1):
                xs_recv(p, h + 1, my_z - h - 1).wait_recv()
                ag_send_h[p][h].wait_send()
            xs_recv(p, 0, my_z).wait_recv()
        for hdl in started:
            hdl.wait_send()

    return pl.pallas_call(
        body,
        out_shape=jax.ShapeDtypeStruct((M, N_OUT), jnp.float32),
        in_specs=[pl.BlockSpec(memory_space=pltpu.VMEM)],
        out_specs=pl.BlockSpec(memory_space=pltpu.VMEM),
        scratch_shapes=[
            pltpu.VMEM((P, Y - 1, RP, NQ), jnp.float32),
            pltpu.VMEM((P, Y - 1, RP, NQ), jnp.float32),
            pltpu.SemaphoreType.DMA((P, Y - 1)),
            pltpu.SemaphoreType.DMA((P, Y - 1)),
            pltpu.SemaphoreType.DMA((P, Z - 1)),
            pltpu.SemaphoreType.DMA((P, Z - 1)),
            pltpu.SemaphoreType.DMA((P, Z)),
            pltpu.SemaphoreType.DMA((P, Z)),
        ],
        compiler_params=pltpu.CompilerParams(collective_id=0),
    )(x)
